# baseline (speedup 1.0000x reference)
import functools

import jax
import jax.numpy as jnp
from jax import lax
from jax.experimental import pallas as pl
from jax.experimental.pallas import tpu as pltpu
from jax.experimental.pallas import tpu_sc as plsc

_F32 = jnp.float32
_I32 = jnp.int32

_E = 16
_NG = 4
_GS = _E // _NG
_T = 2048
_H = 2048
_BT = 256
_NBLK = 32
_NTOT = _NBLK * _BT
_NW = 16
_TW = _T // _NW
_RW = _NTOT // _NW
_GCH = 32


def _iota16():
    return lax.broadcasted_iota(_I32, (16,), 0)


def _splat(val, dtype=_I32):
    return jnp.full((16,), val, dtype)


def _logits_kernel(x_ref, gw_ref, out_ref):
    out_ref[...] = jnp.dot(x_ref[...], gw_ref[...], preferred_element_type=_F32)


def _sc_route_body(lg_hbm, st_hbm, ws_hbm, be_hbm, pos_hbm,
                   lg_v, e1i_v, e2i_v, w1_v, w2_v, cur_v, bc_v, hall_v,
                   spos_v, spos2_v, stok_v, swv_v, pos1_v, pos2_v, bev_v,
                   zz_v, wsz_v, ptmp_v,
                   hist_sh, stok_sh, wsrt_sh):
    wid = lax.axis_index("s")
    t0 = wid * _TW
    lanes = _iota16()

    ptmp_v[pl.ds(0, 16)] = jnp.zeros((16,), _I32)

    def psum16(x):
        acc = x
        for k in (1, 2, 4, 8):
            ptmp_v[pl.ds(16, 16)] = acc
            acc = acc + ptmp_v[pl.ds(16 - k, 16)]
        return acc

    pltpu.sync_copy(lg_hbm.at[:, pl.ds(t0, _TW)], lg_v)

    def route_group(g, accs):
        s = [lg_v[e, pl.ds(g * 16, 16)] for e in range(_E)]
        m = s[0]
        for e in range(1, _E):
            m = jnp.maximum(m, s[e])
        ex = [jnp.exp(s[e] - m) for e in range(_E)]
        tot = ex[0]
        for e in range(1, _E):
            tot = tot + ex[e]
        inv = 1.0 / tot
        sc = [ex[e] * inv for e in range(_E)]
        gm = []
        for gi in range(_NG):
            v = sc[gi * _GS]
            for j in range(1, _GS):
                v = jnp.maximum(v, sc[gi * _GS + j])
            gm.append(v)
        v1 = gm[0]
        i1 = jnp.zeros((16,), _I32)
        for gi in range(1, _NG):
            better = gm[gi] > v1
            v1 = jnp.where(better, gm[gi], v1)
            i1 = jnp.where(better, gi, i1)
        v2 = _splat(-1e30, _F32)
        i2 = _splat(-1, _I32)
        for gi in range(_NG):
            cand = (gm[gi] > v2) & (i1 != gi)
            v2 = jnp.where(cand, gm[gi], v2)
            i2 = jnp.where(cand, gi, i2)
        ms = []
        e1v = _splat(-1e30, _F32)
        e1i = jnp.zeros((16,), _I32)
        for e in range(_E):
            ge = e // _GS
            allowed = (i1 == ge) | (i2 == ge)
            mse = jnp.where(allowed, sc[e], 0.0)
            ms.append(mse)
            better = mse > e1v
            e1v = jnp.where(better, mse, e1v)
            e1i = jnp.where(better, e, e1i)
        e2v = _splat(-1e30, _F32)
        e2i = jnp.zeros((16,), _I32)
        for e in range(_E):
            cand = (ms[e] > e2v) & (e1i != e)
            e2v = jnp.where(cand, ms[e], e2v)
            e2i = jnp.where(cand, e, e2i)
        den = e1v + e2v + 1e-20
        w1 = e1v / den
        w2 = e2v / den
        e1i_v[pl.ds(g * 16, 16)] = e1i
        e2i_v[pl.ds(g * 16, 16)] = e2i
        w1_v[pl.ds(g * 16, 16)] = w1
        w2_v[pl.ds(g * 16, 16)] = w2
        return tuple(
            accs[e]
            + jnp.where(e1i == e, 1, 0)
            + jnp.where(e2i == e, 1, 0)
            for e in range(_E)
        )

    accs = lax.fori_loop(0, _TW // 16, route_group,
                         tuple(jnp.zeros((16,), _I32) for _ in range(_E)))
    hist = jnp.zeros((16,), _I32)
    for e in range(_E):
        tot_e = psum16(accs[e])[15]
        hist = hist + jnp.where(lanes == e, tot_e, 0)

    bc_v[...] = hist
    pltpu.sync_copy(bc_v, hist_sh.at[pl.ds(wid * 16, 16)])
    zi = jnp.zeros((16,), _I32)
    zf = jnp.zeros((16,), _F32)
    for j in range(_RW // 16):
        zz_v[pl.ds(j * 16, 16)] = zi
        wsz_v[pl.ds(j * 16, 16)] = zf
    pltpu.sync_copy(zz_v, stok_sh.at[pl.ds(wid * _RW, _RW)])
    pltpu.sync_copy(wsz_v, wsrt_sh.at[pl.ds(wid * _RW, _RW)])
    plsc.subcore_barrier()
    pltpu.sync_copy(hist_sh, hall_v)
    totals = jnp.zeros((16,), _I32)
    prefix = jnp.zeros((16,), _I32)
    for w in range(_NW):
        row = hall_v[pl.ds(w * 16, 16)]
        totals = totals + row
        prefix = prefix + row * (jnp.int32(w) < wid).astype(_I32)
    pad = (totals + (_BT - 1)) & _splat(-_BT)
    bb_incl = psum16(pad)
    block_base = bb_incl - pad
    cur_v[...] = block_base + prefix

    @pl.when(wid == 0)
    def _():
        endvec = lax.shift_right_logical(bb_incl, 8)
        blk0 = lanes
        blk1 = lanes + 16
        acc0 = jnp.zeros((16,), _I32)
        acc1 = jnp.zeros((16,), _I32)
        for e in range(_E):
            endv = endvec[e]
            acc0 = acc0 + jnp.where(blk0 >= endv, 1, 0)
            acc1 = acc1 + jnp.where(blk1 >= endv, 1, 0)
        bev_v[pl.ds(0, 16)] = jnp.minimum(acc0, _E - 1)
        bev_v[pl.ds(16, 16)] = jnp.minimum(acc1, _E - 1)
        pltpu.sync_copy(bev_v, be_hbm)

    def place(ids, pos_init):
        pos = pos_init
        for e in range(_E):
            msk = ids == e
            rank = psum16(jnp.where(msk, 1, 0))
            cnt = rank[15]
            cur = cur_v[...]
            base = cur[e]
            pos = jnp.where(msk, base + rank - 1, pos)
            cur_v[...] = cur + jnp.where(lanes == e, cnt, 0)
        return pos

    def place_group(g, carry):
        ids1 = e1i_v[pl.ds(g * 16, 16)]
        ids2 = e2i_v[pl.ds(g * 16, 16)]
        tok = t0 + g * 16 + lanes
        p1 = place(ids1, jnp.zeros((16,), _I32))
        p2 = place(ids2, jnp.zeros((16,), _I32))
        pos1_v[pl.ds(g * 16, 16)] = p1
        pos2_v[pl.ds(g * 16, 16)] = p2
        spos_v[pl.ds(g * 32, 16)] = p1
        spos_v[pl.ds(g * 32 + 16, 16)] = p2
        stok_v[pl.ds(g * 32, 16)] = tok
        stok_v[pl.ds(g * 32 + 16, 16)] = tok
        swv_v[pl.ds(g * 32, 16)] = w1_v[pl.ds(g * 16, 16)]
        swv_v[pl.ds(g * 32 + 16, 16)] = w2_v[pl.ds(g * 16, 16)]
        return carry

    lax.fori_loop(0, _TW // 16, place_group, 0)

    for j in range(16):
        spos2_v[j // 8, pl.ds((j % 8) * 16, 16)] = spos_v[pl.ds(j * 16, 16)]
    for k in range(2):
        pltpu.sync_copy(stok_v.at[pl.ds(k * 128, 128)],
                        stok_sh.at[spos2_v.at[k]])
        pltpu.sync_copy(swv_v.at[pl.ds(k * 128, 128)],
                        wsrt_sh.at[spos2_v.at[k]])
    pltpu.sync_copy(pos1_v, pos_hbm.at[0, pl.ds(t0, _TW)])
    pltpu.sync_copy(pos2_v, pos_hbm.at[1, pl.ds(t0, _TW)])
    plsc.subcore_barrier()

    r0 = wid * _RW
    pltpu.sync_copy(stok_sh.at[pl.ds(r0, _RW)], st_hbm.at[pl.ds(r0, _RW)])
    pltpu.sync_copy(wsrt_sh.at[pl.ds(r0, _RW)], ws_hbm.at[pl.ds(r0, _RW)])


_GNW = 32
_GRW = _NTOT // _GNW
_GNB = 3


def _sc_gather_body(st_hbm, x_hbm, xs_hbm, sidx_v, rows0_v, rows1_v, rows2_v,
                    sem0, sem1, sem2):
    wid = lax.axis_index("s") * 2 + lax.axis_index("c")
    r0 = wid * _GRW
    pltpu.sync_copy(st_hbm.at[pl.ds(r0, _GRW)], sidx_v)
    bufs = (rows0_v, rows1_v, rows2_v)
    sems = (sem0, sem1, sem2)
    nch = _GRW // _GCH
    pend = []
    for c in range(min(_GNB, nch)):
        pend.append(pltpu.async_copy(
            x_hbm.at[sidx_v.at[pl.ds(c * _GCH, _GCH)]],
            bufs[c % _GNB], sems[c % _GNB]))
    for c in range(nch):
        pend[c].wait()
        pltpu.sync_copy(bufs[c % _GNB],
                        xs_hbm.at[pl.ds(r0 + c * _GCH, _GCH), :])
        if c + _GNB < nch:
            pend.append(pltpu.async_copy(
                x_hbm.at[sidx_v.at[pl.ds((c + _GNB) * _GCH, _GCH)]],
                bufs[(c + _GNB) % _GNB], sems[(c + _GNB) % _GNB]))


def _gmm_kernel(be_ref, xs_ref, ws_ref, wg_ref, wu_ref, wd_ref, y_ref,
                h_acc, u_acc):
    kc = pl.program_id(1)
    xb = xs_ref[...]
    hp = jnp.dot(xb, wg_ref[0], preferred_element_type=_F32)
    up = jnp.dot(xb, wu_ref[0], preferred_element_type=_F32)

    @pl.when(kc == 0)
    def _():
        h_acc[...] = hp
        u_acc[...] = up

    @pl.when(kc == 1)
    def _():
        h = h_acc[...] + hp
        u = u_acc[...] + up
        a = jax.nn.silu(h) * u * ws_ref[...]
        y_ref[...] = jnp.dot(a, wd_ref[0], preferred_element_type=_F32)


def _shared_kernel(x_ref, g_ref, u_ref, d_ref, out_ref):
    fs = pl.program_id(1)
    xb = x_ref[...]
    h = jnp.dot(xb, g_ref[...], preferred_element_type=_F32)
    u = jnp.dot(xb, u_ref[...], preferred_element_type=_F32)
    a = (jax.nn.silu(h) * u).astype(jnp.bfloat16)
    val = jnp.dot(a, d_ref[...], preferred_element_type=_F32)

    @pl.when(fs == 0)
    def _():
        out_ref[...] = val

    @pl.when(fs > 0)
    def _():
        out_ref[...] += val


_CNW = 32
_CTW = _T // _CNW
_CCH = 16


def _sc_combine_body(y_hbm, sh_hbm, pos_hbm, out_hbm,
                     p1_v, p2_v, b1_v, b2_v, b3_v, sem1, sem2):
    wid = lax.axis_index("s") * 2 + lax.axis_index("c")
    t0 = wid * _CTW
    pltpu.sync_copy(pos_hbm.at[0, pl.ds(t0, _CTW)], p1_v)
    pltpu.sync_copy(pos_hbm.at[1, pl.ds(t0, _CTW)], p2_v)

    def chunk(c, carry):
        g1 = pltpu.async_copy(y_hbm.at[p1_v.at[pl.ds(c * _CCH, _CCH)]],
                              b1_v, sem1)
        g2 = pltpu.async_copy(y_hbm.at[p2_v.at[pl.ds(c * _CCH, _CCH)]],
                              b2_v, sem2)
        pltpu.sync_copy(sh_hbm.at[pl.ds(t0 + c * _CCH, _CCH), :], b3_v)
        g1.wait()
        g2.wait()
        for r in range(_CCH):
            def inner(j, carry2):
                col = j * 64
                for u in range(4):
                    sl = pl.ds(col + u * 16, 16)
                    b1_v[r, sl] = b1_v[r, sl] + b2_v[r, sl] + b3_v[r, sl]
                return carry2
            lax.fori_loop(0, _H // 64, inner, 0)
        pltpu.sync_copy(b1_v, out_hbm.at[pl.ds(t0 + c * _CCH, _CCH), :])
        return carry

    lax.fori_loop(0, _CTW // _CCH, chunk, 0)


def kernel(hidden_states, gate_w, w_gate, w_up, w_down, sw_gate, sw_up, sw_down):
    x = hidden_states
    T, H = x.shape
    E = gate_w.shape[1]
    F = w_gate.shape[2]
    DS = sw_gate.shape[1]

    logits = pl.pallas_call(
        _logits_kernel,
        grid=(T // 256,),
        in_specs=[
            pl.BlockSpec((256, H), lambda i: (i, 0)),
            pl.BlockSpec((H, E), lambda i: (0, 0)),
        ],
        out_specs=pl.BlockSpec((256, E), lambda i: (i, 0)),
        out_shape=jax.ShapeDtypeStruct((T, E), _F32),
    )(x, gate_w)

    route = pl.kernel(
        _sc_route_body,
        out_type=[
            jax.ShapeDtypeStruct((_NTOT,), _I32),
            jax.ShapeDtypeStruct((_NTOT,), _F32),
            jax.ShapeDtypeStruct((_NBLK,), _I32),
            jax.ShapeDtypeStruct((2, T), _I32),
        ],
        mesh=plsc.VectorSubcoreMesh(core_axis_name="c", subcore_axis_name="s",
                                    num_cores=1),
        scratch_types=[
            pltpu.VMEM((_E, _TW), _F32),
            pltpu.VMEM((_TW,), _I32),
            pltpu.VMEM((_TW,), _I32),
            pltpu.VMEM((_TW,), _F32),
            pltpu.VMEM((_TW,), _F32),
            pltpu.VMEM((16,), _I32),
            pltpu.VMEM((16,), _I32),
            pltpu.VMEM((_NW * 16,), _I32),
            pltpu.VMEM((2 * _TW,), _I32),
            pltpu.VMEM((2, 128), _I32),
            pltpu.VMEM((2 * _TW,), _I32),
            pltpu.VMEM((2 * _TW,), _F32),
            pltpu.VMEM((_TW,), _I32),
            pltpu.VMEM((_TW,), _I32),
            pltpu.VMEM((_NBLK,), _I32),
            pltpu.VMEM((_RW,), _I32),
            pltpu.VMEM((_RW,), _F32),
            pltpu.VMEM((32,), _I32),
            pltpu.VMEM_SHARED((_NW * 16,), _I32),
            pltpu.VMEM_SHARED((_NTOT,), _I32),
            pltpu.VMEM_SHARED((_NTOT,), _F32),
        ],
    )
    st, ws, be, pos = route(logits.T)

    bf16 = jnp.bfloat16
    x16 = x.astype(bf16)
    x32 = lax.bitcast_convert_type(x16.reshape(T, H // 2, 2), _I32)
    gather = pl.kernel(
        _sc_gather_body,
        out_type=jax.ShapeDtypeStruct((_NTOT, H // 2), _I32),
        mesh=plsc.VectorSubcoreMesh(core_axis_name="c", subcore_axis_name="s"),
        scratch_types=[
            pltpu.VMEM((_GRW,), _I32),
            pltpu.VMEM((_GCH, H // 2), _I32),
            pltpu.VMEM((_GCH, H // 2), _I32),
            pltpu.VMEM((_GCH, H // 2), _I32),
            pltpu.SemaphoreType.DMA,
            pltpu.SemaphoreType.DMA,
            pltpu.SemaphoreType.DMA,
        ],
    )
    xs32 = gather(st, x32)
    xs = lax.bitcast_convert_type(xs32, bf16).reshape(_NTOT, H)

    fs = 1408 if DS % 1408 == 0 else DS
    shared = pl.pallas_call(
        _shared_kernel,
        grid=(T // 512, DS // fs),
        in_specs=[
            pl.BlockSpec((512, H), lambda i, j: (i, 0)),
            pl.BlockSpec((H, fs), lambda i, j: (0, j)),
            pl.BlockSpec((H, fs), lambda i, j: (0, j)),
            pl.BlockSpec((fs, H), lambda i, j: (j, 0)),
        ],
        out_specs=pl.BlockSpec((512, H), lambda i, j: (i, 0)),
        out_shape=jax.ShapeDtypeStruct((T, H), _F32),
    )(x16, sw_gate.astype(bf16), sw_up.astype(bf16), sw_down.astype(bf16))

    kcn = 2
    kch = H // kcn
    y = pl.pallas_call(
        _gmm_kernel,
        grid_spec=pltpu.PrefetchScalarGridSpec(
            num_scalar_prefetch=1,
            grid=(_NBLK, kcn),
            in_specs=[
                pl.BlockSpec((_BT, kch), lambda b, k, be: (b, k)),
                pl.BlockSpec((_BT, 1), lambda b, k, be: (b, 0)),
                pl.BlockSpec((1, kch, F), lambda b, k, be: (be[b], k, 0)),
                pl.BlockSpec((1, kch, F), lambda b, k, be: (be[b], k, 0)),
                pl.BlockSpec((1, F, H), lambda b, k, be: (be[b], 0, 0)),
            ],
            out_specs=pl.BlockSpec((_BT, H), lambda b, k, be: (b, 0)),
            scratch_shapes=[
                pltpu.VMEM((_BT, F), _F32),
                pltpu.VMEM((_BT, F), _F32),
            ],
        ),
        out_shape=jax.ShapeDtypeStruct((_NTOT, H), _F32),
    )(be, xs, ws.reshape(_NTOT, 1), w_gate, w_up, w_down)

    comb = pl.kernel(
        _sc_combine_body,
        out_type=jax.ShapeDtypeStruct((T, H), _F32),
        mesh=plsc.VectorSubcoreMesh(core_axis_name="c", subcore_axis_name="s"),
        scratch_types=[
            pltpu.VMEM((_CTW,), _I32),
            pltpu.VMEM((_CTW,), _I32),
            pltpu.VMEM((_CCH, H), _F32),
            pltpu.VMEM((_CCH, H), _F32),
            pltpu.VMEM((_CCH, H), _F32),
            pltpu.SemaphoreType.DMA,
            pltpu.SemaphoreType.DMA,
        ],
    )
    out = comb(y, shared, pos)
    return out


# speedup vs baseline: 1.3643x; 1.3643x over previous
import functools

import jax
import jax.numpy as jnp
from jax import lax
from jax.experimental import pallas as pl
from jax.experimental.pallas import tpu as pltpu
from jax.experimental.pallas import tpu_sc as plsc

_F32 = jnp.float32
_I32 = jnp.int32

_E = 16
_NG = 4
_GS = _E // _NG
_T = 2048
_H = 2048
_BT = 256
_NBLK = 32
_NTOT = _NBLK * _BT
_NW = 16
_TW = _T // _NW
_RW = _NTOT // _NW
_GCH = 16


def _iota16():
    return lax.broadcasted_iota(_I32, (16,), 0)


def _splat(val, dtype=_I32):
    return jnp.full((16,), val, dtype)


def _logits_kernel(x_ref, gw_ref, out_ref):
    out_ref[...] = jnp.dot(x_ref[...], gw_ref[...], preferred_element_type=_F32)


def _sc_route_body(lg_hbm, st_hbm, ws_hbm, be_hbm, pos_hbm,
                   lg_v, e1i_v, e2i_v, w1_v, w2_v, cur_v, bc_v, hall_v,
                   spos_v, spos2_v, stok_v, swv_v, pos1_v, pos2_v, bev_v,
                   zz_v, wsz_v, ptmp_v,
                   hist_sh, stok_sh, wsrt_sh):
    wid = lax.axis_index("s")
    t0 = wid * _TW
    lanes = _iota16()

    ptmp_v[pl.ds(0, 16)] = jnp.zeros((16,), _I32)

    def psum16(x):
        acc = x
        for k in (1, 2, 4, 8):
            ptmp_v[pl.ds(16, 16)] = acc
            acc = acc + ptmp_v[pl.ds(16 - k, 16)]
        return acc

    pltpu.sync_copy(lg_hbm.at[:, pl.ds(t0, _TW)], lg_v)

    def route_group(g, accs):
        s = [lg_v[e, pl.ds(g * 16, 16)] for e in range(_E)]
        m = s[0]
        for e in range(1, _E):
            m = jnp.maximum(m, s[e])
        ex = [jnp.exp(s[e] - m) for e in range(_E)]
        tot = ex[0]
        for e in range(1, _E):
            tot = tot + ex[e]
        inv = 1.0 / tot
        sc = [ex[e] * inv for e in range(_E)]
        gm = []
        for gi in range(_NG):
            v = sc[gi * _GS]
            for j in range(1, _GS):
                v = jnp.maximum(v, sc[gi * _GS + j])
            gm.append(v)
        v1 = gm[0]
        i1 = jnp.zeros((16,), _I32)
        for gi in range(1, _NG):
            better = gm[gi] > v1
            v1 = jnp.where(better, gm[gi], v1)
            i1 = jnp.where(better, gi, i1)
        v2 = _splat(-1e30, _F32)
        i2 = _splat(-1, _I32)
        for gi in range(_NG):
            cand = (gm[gi] > v2) & (i1 != gi)
            v2 = jnp.where(cand, gm[gi], v2)
            i2 = jnp.where(cand, gi, i2)
        ms = []
        e1v = _splat(-1e30, _F32)
        e1i = jnp.zeros((16,), _I32)
        for e in range(_E):
            ge = e // _GS
            allowed = (i1 == ge) | (i2 == ge)
            mse = jnp.where(allowed, sc[e], 0.0)
            ms.append(mse)
            better = mse > e1v
            e1v = jnp.where(better, mse, e1v)
            e1i = jnp.where(better, e, e1i)
        e2v = _splat(-1e30, _F32)
        e2i = jnp.zeros((16,), _I32)
        for e in range(_E):
            cand = (ms[e] > e2v) & (e1i != e)
            e2v = jnp.where(cand, ms[e], e2v)
            e2i = jnp.where(cand, e, e2i)
        den = e1v + e2v + 1e-20
        w1 = e1v / den
        w2 = e2v / den
        e1i_v[pl.ds(g * 16, 16)] = e1i
        e2i_v[pl.ds(g * 16, 16)] = e2i
        w1_v[pl.ds(g * 16, 16)] = w1
        w2_v[pl.ds(g * 16, 16)] = w2
        return tuple(
            accs[e]
            + jnp.where(e1i == e, 1, 0)
            + jnp.where(e2i == e, 1, 0)
            for e in range(_E)
        )

    accs = lax.fori_loop(0, _TW // 16, route_group,
                         tuple(jnp.zeros((16,), _I32) for _ in range(_E)))
    hist = jnp.zeros((16,), _I32)
    for e in range(_E):
        tot_e = psum16(accs[e])[15]
        hist = hist + jnp.where(lanes == e, tot_e, 0)

    bc_v[...] = hist
    pltpu.sync_copy(bc_v, hist_sh.at[pl.ds(wid * 16, 16)])
    zi = jnp.zeros((16,), _I32)
    zf = jnp.zeros((16,), _F32)
    for j in range(_RW // 16):
        zz_v[pl.ds(j * 16, 16)] = zi
        wsz_v[pl.ds(j * 16, 16)] = zf
    pltpu.sync_copy(zz_v, stok_sh.at[pl.ds(wid * _RW, _RW)])
    pltpu.sync_copy(wsz_v, wsrt_sh.at[pl.ds(wid * _RW, _RW)])
    plsc.subcore_barrier()
    pltpu.sync_copy(hist_sh, hall_v)
    totals = jnp.zeros((16,), _I32)
    prefix = jnp.zeros((16,), _I32)
    for w in range(_NW):
        row = hall_v[pl.ds(w * 16, 16)]
        totals = totals + row
        prefix = prefix + row * (jnp.int32(w) < wid).astype(_I32)
    pad = (totals + (_BT - 1)) & _splat(-_BT)
    bb_incl = psum16(pad)
    block_base = bb_incl - pad
    cur_v[...] = block_base + prefix

    @pl.when(wid == 0)
    def _():
        endvec = lax.shift_right_logical(bb_incl, 8)
        blk0 = lanes
        blk1 = lanes + 16
        acc0 = jnp.zeros((16,), _I32)
        acc1 = jnp.zeros((16,), _I32)
        for e in range(_E):
            endv = endvec[e]
            acc0 = acc0 + jnp.where(blk0 >= endv, 1, 0)
            acc1 = acc1 + jnp.where(blk1 >= endv, 1, 0)
        bev_v[pl.ds(0, 16)] = jnp.minimum(acc0, _E - 1)
        bev_v[pl.ds(16, 16)] = jnp.minimum(acc1, _E - 1)
        pltpu.sync_copy(bev_v, be_hbm)

    def place(ids, pos_init):
        pos = pos_init
        for e in range(_E):
            msk = ids == e
            rank = psum16(jnp.where(msk, 1, 0))
            cnt = rank[15]
            cur = cur_v[...]
            base = cur[e]
            pos = jnp.where(msk, base + rank - 1, pos)
            cur_v[...] = cur + jnp.where(lanes == e, cnt, 0)
        return pos

    def place_group(g, carry):
        ids1 = e1i_v[pl.ds(g * 16, 16)]
        ids2 = e2i_v[pl.ds(g * 16, 16)]
        tok = t0 + g * 16 + lanes
        p1 = place(ids1, jnp.zeros((16,), _I32))
        p2 = place(ids2, jnp.zeros((16,), _I32))
        pos1_v[pl.ds(g * 16, 16)] = p1
        pos2_v[pl.ds(g * 16, 16)] = p2
        spos_v[pl.ds(g * 32, 16)] = p1
        spos_v[pl.ds(g * 32 + 16, 16)] = p2
        stok_v[pl.ds(g * 32, 16)] = tok
        stok_v[pl.ds(g * 32 + 16, 16)] = tok
        swv_v[pl.ds(g * 32, 16)] = w1_v[pl.ds(g * 16, 16)]
        swv_v[pl.ds(g * 32 + 16, 16)] = w2_v[pl.ds(g * 16, 16)]
        return carry

    lax.fori_loop(0, _TW // 16, place_group, 0)

    for j in range(16):
        spos2_v[j // 8, pl.ds((j % 8) * 16, 16)] = spos_v[pl.ds(j * 16, 16)]
    for k in range(2):
        pltpu.sync_copy(stok_v.at[pl.ds(k * 128, 128)],
                        stok_sh.at[spos2_v.at[k]])
        pltpu.sync_copy(swv_v.at[pl.ds(k * 128, 128)],
                        wsrt_sh.at[spos2_v.at[k]])
    pltpu.sync_copy(pos1_v, pos_hbm.at[0, pl.ds(t0, _TW)])
    pltpu.sync_copy(pos2_v, pos_hbm.at[1, pl.ds(t0, _TW)])
    plsc.subcore_barrier()

    r0 = wid * _RW
    pltpu.sync_copy(stok_sh.at[pl.ds(r0, _RW)], st_hbm.at[pl.ds(r0, _RW)])
    pltpu.sync_copy(wsrt_sh.at[pl.ds(r0, _RW)], ws_hbm.at[pl.ds(r0, _RW)])


_GNW = 32
_GRW = _NTOT // _GNW
_GNB = 3


def _sc_gather_body(st_hbm, x_hbm, xs_hbm, sidx_v, rows0_v, rows1_v, rows2_v,
                    sem0, sem1, sem2):
    wid = lax.axis_index("s") * 2 + lax.axis_index("c")
    r0 = wid * _GRW
    pltpu.sync_copy(st_hbm.at[pl.ds(r0, _GRW)], sidx_v)
    bufs = (rows0_v, rows1_v, rows2_v)
    sems = (sem0, sem1, sem2)
    nch = _GRW // _GCH
    pend = []
    for c in range(min(_GNB, nch)):
        pend.append(pltpu.async_copy(
            x_hbm.at[sidx_v.at[pl.ds(c * _GCH, _GCH)]],
            bufs[c % _GNB], sems[c % _GNB]))
    for c in range(nch):
        pend[c].wait()
        pltpu.sync_copy(bufs[c % _GNB],
                        xs_hbm.at[pl.ds(r0 + c * _GCH, _GCH), :])
        if c + _GNB < nch:
            pend.append(pltpu.async_copy(
                x_hbm.at[sidx_v.at[pl.ds((c + _GNB) * _GCH, _GCH)]],
                bufs[(c + _GNB) % _GNB], sems[(c + _GNB) % _GNB]))


def _gmm_kernel(be_ref, xs_ref, ws_ref, wg_ref, wu_ref, wd_ref, y_ref,
                h_acc, u_acc):
    kc = pl.program_id(1)
    xb = xs_ref[...]
    hp = jnp.dot(xb, wg_ref[0], preferred_element_type=_F32)
    up = jnp.dot(xb, wu_ref[0], preferred_element_type=_F32)

    @pl.when(kc == 0)
    def _():
        h_acc[...] = hp
        u_acc[...] = up

    @pl.when(kc == 1)
    def _():
        h = h_acc[...] + hp
        u = u_acc[...] + up
        a = jax.nn.silu(h) * u * ws_ref[...]
        y_ref[...] = jnp.dot(a, wd_ref[0], preferred_element_type=_F32)


def _shared_kernel(x_ref, g_ref, u_ref, d_ref, out_ref):
    fs = pl.program_id(1)
    xb = x_ref[...].astype(jnp.bfloat16)
    h = jnp.dot(xb, g_ref[...], preferred_element_type=_F32)
    u = jnp.dot(xb, u_ref[...], preferred_element_type=_F32)
    a = (jax.nn.silu(h) * u).astype(jnp.bfloat16)
    val = jnp.dot(a, d_ref[...], preferred_element_type=_F32)

    @pl.when(fs == 0)
    def _():
        out_ref[...] = val

    @pl.when(fs > 0)
    def _():
        out_ref[...] += val


_CNW = 32
_CTW = _T // _CNW
_CCH = 16


def _sc_combine_body(y_hbm, sh_hbm, pos_hbm, out_hbm,
                     p1_v, p2_v, b1_v, b2_v, b3_v, sem1, sem2):
    wid = lax.axis_index("s") * 2 + lax.axis_index("c")
    t0 = wid * _CTW
    pltpu.sync_copy(pos_hbm.at[0, pl.ds(t0, _CTW)], p1_v)
    pltpu.sync_copy(pos_hbm.at[1, pl.ds(t0, _CTW)], p2_v)

    def chunk(c, carry):
        g1 = pltpu.async_copy(y_hbm.at[p1_v.at[pl.ds(c * _CCH, _CCH)]],
                              b1_v, sem1)
        g2 = pltpu.async_copy(y_hbm.at[p2_v.at[pl.ds(c * _CCH, _CCH)]],
                              b2_v, sem2)
        pltpu.sync_copy(sh_hbm.at[pl.ds(t0 + c * _CCH, _CCH), :], b3_v)
        g1.wait()
        g2.wait()
        for r in range(_CCH):
            def inner(j, carry2):
                col = j * 64
                for u in range(4):
                    sl = pl.ds(col + u * 16, 16)
                    b1_v[r, sl] = b1_v[r, sl] + b2_v[r, sl] + b3_v[r, sl]
                return carry2
            lax.fori_loop(0, _H // 64, inner, 0)
        pltpu.sync_copy(b1_v, out_hbm.at[pl.ds(t0 + c * _CCH, _CCH), :])
        return carry

    lax.fori_loop(0, _CTW // _CCH, chunk, 0)


def kernel(hidden_states, gate_w, w_gate, w_up, w_down, sw_gate, sw_up, sw_down):
    x = hidden_states
    T, H = x.shape
    E = gate_w.shape[1]
    F = w_gate.shape[2]
    DS = sw_gate.shape[1]

    logits = pl.pallas_call(
        _logits_kernel,
        grid=(T // 256,),
        in_specs=[
            pl.BlockSpec((256, H), lambda i: (i, 0)),
            pl.BlockSpec((H, E), lambda i: (0, 0)),
        ],
        out_specs=pl.BlockSpec((256, E), lambda i: (i, 0)),
        out_shape=jax.ShapeDtypeStruct((T, E), _F32),
    )(x, gate_w)

    bf16 = jnp.bfloat16
    fs = 1408 if DS % 1408 == 0 else DS
    shared = pl.pallas_call(
        _shared_kernel,
        grid=(T // 512, DS // fs),
        in_specs=[
            pl.BlockSpec((512, H), lambda i, j: (i, 0)),
            pl.BlockSpec((H, fs), lambda i, j: (0, j)),
            pl.BlockSpec((H, fs), lambda i, j: (0, j)),
            pl.BlockSpec((fs, H), lambda i, j: (j, 0)),
        ],
        out_specs=pl.BlockSpec((512, H), lambda i, j: (i, 0)),
        out_shape=jax.ShapeDtypeStruct((T, H), _F32),
    )(x, sw_gate.astype(bf16), sw_up.astype(bf16), sw_down.astype(bf16))

    route = pl.kernel(
        _sc_route_body,
        out_type=[
            jax.ShapeDtypeStruct((_NTOT,), _I32),
            jax.ShapeDtypeStruct((_NTOT,), _F32),
            jax.ShapeDtypeStruct((_NBLK,), _I32),
            jax.ShapeDtypeStruct((2, T), _I32),
        ],
        mesh=plsc.VectorSubcoreMesh(core_axis_name="c", subcore_axis_name="s",
                                    num_cores=1),
        scratch_types=[
            pltpu.VMEM((_E, _TW), _F32),
            pltpu.VMEM((_TW,), _I32),
            pltpu.VMEM((_TW,), _I32),
            pltpu.VMEM((_TW,), _F32),
            pltpu.VMEM((_TW,), _F32),
            pltpu.VMEM((16,), _I32),
            pltpu.VMEM((16,), _I32),
            pltpu.VMEM((_NW * 16,), _I32),
            pltpu.VMEM((2 * _TW,), _I32),
            pltpu.VMEM((2, 128), _I32),
            pltpu.VMEM((2 * _TW,), _I32),
            pltpu.VMEM((2 * _TW,), _F32),
            pltpu.VMEM((_TW,), _I32),
            pltpu.VMEM((_TW,), _I32),
            pltpu.VMEM((_NBLK,), _I32),
            pltpu.VMEM((_RW,), _I32),
            pltpu.VMEM((_RW,), _F32),
            pltpu.VMEM((32,), _I32),
            pltpu.VMEM_SHARED((_NW * 16,), _I32),
            pltpu.VMEM_SHARED((_NTOT,), _I32),
            pltpu.VMEM_SHARED((_NTOT,), _F32),
        ],
    )
    st, ws, be, pos = route(logits.T)

    gather = pl.kernel(
        _sc_gather_body,
        out_type=jax.ShapeDtypeStruct((_NTOT, H), _F32),
        mesh=plsc.VectorSubcoreMesh(core_axis_name="c", subcore_axis_name="s"),
        scratch_types=[
            pltpu.VMEM((_GRW,), _I32),
            pltpu.VMEM((_GCH, H), _F32),
            pltpu.VMEM((_GCH, H), _F32),
            pltpu.VMEM((_GCH, H), _F32),
            pltpu.SemaphoreType.DMA,
            pltpu.SemaphoreType.DMA,
            pltpu.SemaphoreType.DMA,
        ],
    )
    xs = gather(st, x)

    kcn = 2
    kch = H // kcn
    y = pl.pallas_call(
        _gmm_kernel,
        grid_spec=pltpu.PrefetchScalarGridSpec(
            num_scalar_prefetch=1,
            grid=(_NBLK, kcn),
            in_specs=[
                pl.BlockSpec((_BT, kch), lambda b, k, be: (b, k)),
                pl.BlockSpec((_BT, 1), lambda b, k, be: (b, 0)),
                pl.BlockSpec((1, kch, F), lambda b, k, be: (be[b], k, 0)),
                pl.BlockSpec((1, kch, F), lambda b, k, be: (be[b], k, 0)),
                pl.BlockSpec((1, F, H), lambda b, k, be: (be[b], 0, 0)),
            ],
            out_specs=pl.BlockSpec((_BT, H), lambda b, k, be: (b, 0)),
            scratch_shapes=[
                pltpu.VMEM((_BT, F), _F32),
                pltpu.VMEM((_BT, F), _F32),
            ],
        ),
        out_shape=jax.ShapeDtypeStruct((_NTOT, H), _F32),
    )(be, xs, ws.reshape(_NTOT, 1), w_gate, w_up, w_down)

    comb = pl.kernel(
        _sc_combine_body,
        out_type=jax.ShapeDtypeStruct((T, H), _F32),
        mesh=plsc.VectorSubcoreMesh(core_axis_name="c", subcore_axis_name="s"),
        scratch_types=[
            pltpu.VMEM((_CTW,), _I32),
            pltpu.VMEM((_CTW,), _I32),
            pltpu.VMEM((_CCH, H), _F32),
            pltpu.VMEM((_CCH, H), _F32),
            pltpu.VMEM((_CCH, H), _F32),
            pltpu.SemaphoreType.DMA,
            pltpu.SemaphoreType.DMA,
        ],
    )
    out = comb(y, shared, pos)
    return out


# speedup vs baseline: 1.3988x; 1.0252x over previous
import functools

import jax
import jax.numpy as jnp
from jax import lax
from jax.experimental import pallas as pl
from jax.experimental.pallas import tpu as pltpu
from jax.experimental.pallas import tpu_sc as plsc

_F32 = jnp.float32
_I32 = jnp.int32

_E = 16
_NG = 4
_GS = _E // _NG
_T = 2048
_H = 2048
_BT = 256
_NBLK = 32
_NTOT = _NBLK * _BT
_NW = 16
_TW = _T // _NW
_RW = _NTOT // _NW
_GCH = 16


def _iota16():
    return lax.broadcasted_iota(_I32, (16,), 0)


def _splat(val, dtype=_I32):
    return jnp.full((16,), val, dtype)


def _logits_kernel(x_ref, gw_ref, out_ref):
    out_ref[...] = jnp.dot(x_ref[...], gw_ref[...], preferred_element_type=_F32)


def _sc_route_body(lg_hbm, x_hbm, xs_hbm, ws_hbm, be_hbm, pos_hbm,
                   lg_v, e1i_v, e2i_v, w1_v, w2_v, cur_v, bc_v, hall_v,
                   spos_v, spos2_v, stok_v, swv_v, pos1_v, pos2_v, bev_v,
                   zz_v, wsz_v, ptmp_v, sidx_v, rows0_v, rows1_v,
                   hist_sh, stok_sh, wsrt_sh, sem0, sem1):
    wid = lax.axis_index("s")
    t0 = wid * _TW
    lanes = _iota16()

    ptmp_v[pl.ds(0, 16)] = jnp.zeros((16,), _I32)

    def psum16(x):
        acc = x
        for k in (1, 2, 4, 8):
            ptmp_v[pl.ds(16, 16)] = acc
            acc = acc + ptmp_v[pl.ds(16 - k, 16)]
        return acc

    pltpu.sync_copy(lg_hbm.at[:, pl.ds(t0, _TW)], lg_v)

    def route_group(g, accs):
        s = [lg_v[e, pl.ds(g * 16, 16)] for e in range(_E)]
        m = s[0]
        for e in range(1, _E):
            m = jnp.maximum(m, s[e])
        ex = [jnp.exp(s[e] - m) for e in range(_E)]
        tot = ex[0]
        for e in range(1, _E):
            tot = tot + ex[e]
        inv = 1.0 / tot
        sc = [ex[e] * inv for e in range(_E)]
        gm = []
        for gi in range(_NG):
            v = sc[gi * _GS]
            for j in range(1, _GS):
                v = jnp.maximum(v, sc[gi * _GS + j])
            gm.append(v)
        v1 = gm[0]
        i1 = jnp.zeros((16,), _I32)
        for gi in range(1, _NG):
            better = gm[gi] > v1
            v1 = jnp.where(better, gm[gi], v1)
            i1 = jnp.where(better, gi, i1)
        v2 = _splat(-1e30, _F32)
        i2 = _splat(-1, _I32)
        for gi in range(_NG):
            cand = (gm[gi] > v2) & (i1 != gi)
            v2 = jnp.where(cand, gm[gi], v2)
            i2 = jnp.where(cand, gi, i2)
        ms = []
        e1v = _splat(-1e30, _F32)
        e1i = jnp.zeros((16,), _I32)
        for e in range(_E):
            ge = e // _GS
            allowed = (i1 == ge) | (i2 == ge)
            mse = jnp.where(allowed, sc[e], 0.0)
            ms.append(mse)
            better = mse > e1v
            e1v = jnp.where(better, mse, e1v)
            e1i = jnp.where(better, e, e1i)
        e2v = _splat(-1e30, _F32)
        e2i = jnp.zeros((16,), _I32)
        for e in range(_E):
            cand = (ms[e] > e2v) & (e1i != e)
            e2v = jnp.where(cand, ms[e], e2v)
            e2i = jnp.where(cand, e, e2i)
        den = e1v + e2v + 1e-20
        w1 = e1v / den
        w2 = e2v / den
        e1i_v[pl.ds(g * 16, 16)] = e1i
        e2i_v[pl.ds(g * 16, 16)] = e2i
        w1_v[pl.ds(g * 16, 16)] = w1
        w2_v[pl.ds(g * 16, 16)] = w2
        return tuple(
            accs[e]
            + jnp.where(e1i == e, 1, 0)
            + jnp.where(e2i == e, 1, 0)
            for e in range(_E)
        )

    accs = lax.fori_loop(0, _TW // 16, route_group,
                         tuple(jnp.zeros((16,), _I32) for _ in range(_E)))
    hist = jnp.zeros((16,), _I32)
    for e in range(_E):
        tot_e = psum16(accs[e])[15]
        hist = hist + jnp.where(lanes == e, tot_e, 0)

    bc_v[...] = hist
    pltpu.sync_copy(bc_v, hist_sh.at[pl.ds(wid * 16, 16)])
    zi = jnp.zeros((16,), _I32)
    zf = jnp.zeros((16,), _F32)
    for j in range(_RW // 16):
        zz_v[pl.ds(j * 16, 16)] = zi
        wsz_v[pl.ds(j * 16, 16)] = zf
    pltpu.sync_copy(zz_v, stok_sh.at[pl.ds(wid * _RW, _RW)])
    pltpu.sync_copy(wsz_v, wsrt_sh.at[pl.ds(wid * _RW, _RW)])
    plsc.subcore_barrier()
    pltpu.sync_copy(hist_sh, hall_v)
    totals = jnp.zeros((16,), _I32)
    prefix = jnp.zeros((16,), _I32)
    for w in range(_NW):
        row = hall_v[pl.ds(w * 16, 16)]
        totals = totals + row
        prefix = prefix + row * (jnp.int32(w) < wid).astype(_I32)
    pad = (totals + (_BT - 1)) & _splat(-_BT)
    bb_incl = psum16(pad)
    block_base = bb_incl - pad
    cur_v[...] = block_base + prefix

    @pl.when(wid == 0)
    def _():
        endvec = lax.shift_right_logical(bb_incl, 8)
        blk0 = lanes
        blk1 = lanes + 16
        acc0 = jnp.zeros((16,), _I32)
        acc1 = jnp.zeros((16,), _I32)
        for e in range(_E):
            endv = endvec[e]
            acc0 = acc0 + jnp.where(blk0 >= endv, 1, 0)
            acc1 = acc1 + jnp.where(blk1 >= endv, 1, 0)
        bev_v[pl.ds(0, 16)] = jnp.minimum(acc0, _E - 1)
        bev_v[pl.ds(16, 16)] = jnp.minimum(acc1, _E - 1)
        pltpu.sync_copy(bev_v, be_hbm)

    def place(ids, pos_init):
        pos = pos_init
        for e in range(_E):
            msk = ids == e
            rank = psum16(jnp.where(msk, 1, 0))
            cnt = rank[15]
            cur = cur_v[...]
            base = cur[e]
            pos = jnp.where(msk, base + rank - 1, pos)
            cur_v[...] = cur + jnp.where(lanes == e, cnt, 0)
        return pos

    def place_group(g, carry):
        ids1 = e1i_v[pl.ds(g * 16, 16)]
        ids2 = e2i_v[pl.ds(g * 16, 16)]
        tok = t0 + g * 16 + lanes
        p1 = place(ids1, jnp.zeros((16,), _I32))
        p2 = place(ids2, jnp.zeros((16,), _I32))
        pos1_v[pl.ds(g * 16, 16)] = p1
        pos2_v[pl.ds(g * 16, 16)] = p2
        spos_v[pl.ds(g * 32, 16)] = p1
        spos_v[pl.ds(g * 32 + 16, 16)] = p2
        stok_v[pl.ds(g * 32, 16)] = tok
        stok_v[pl.ds(g * 32 + 16, 16)] = tok
        swv_v[pl.ds(g * 32, 16)] = w1_v[pl.ds(g * 16, 16)]
        swv_v[pl.ds(g * 32 + 16, 16)] = w2_v[pl.ds(g * 16, 16)]
        return carry

    lax.fori_loop(0, _TW // 16, place_group, 0)

    for j in range(16):
        spos2_v[j // 8, pl.ds((j % 8) * 16, 16)] = spos_v[pl.ds(j * 16, 16)]
    for k in range(2):
        pltpu.sync_copy(stok_v.at[pl.ds(k * 128, 128)],
                        stok_sh.at[spos2_v.at[k]])
        pltpu.sync_copy(swv_v.at[pl.ds(k * 128, 128)],
                        wsrt_sh.at[spos2_v.at[k]])
    pltpu.sync_copy(pos1_v, pos_hbm.at[0, pl.ds(t0, _TW)])
    pltpu.sync_copy(pos2_v, pos_hbm.at[1, pl.ds(t0, _TW)])
    plsc.subcore_barrier()

    r0 = wid * _RW
    pltpu.sync_copy(stok_sh.at[pl.ds(r0, _RW)], sidx_v)
    pltpu.sync_copy(wsrt_sh.at[pl.ds(r0, _RW)], ws_hbm.at[pl.ds(r0, _RW)])
    bufs = (rows0_v, rows1_v)
    sems = (sem0, sem1)
    nch = _RW // _GCH
    pend = []
    for c in range(2):
        pend.append(pltpu.async_copy(
            x_hbm.at[sidx_v.at[pl.ds(c * _GCH, _GCH)]],
            bufs[c % 2], sems[c % 2]))
    for c in range(nch):
        pend[c].wait()
        pltpu.sync_copy(bufs[c % 2],
                        xs_hbm.at[pl.ds(r0 + c * _GCH, _GCH), :])
        if c + 2 < nch:
            pend.append(pltpu.async_copy(
                x_hbm.at[sidx_v.at[pl.ds((c + 2) * _GCH, _GCH)]],
                bufs[c % 2], sems[c % 2]))


def _gmm_kernel(be_ref, xs_ref, ws_ref, wg_ref, wu_ref, wd_ref, y_ref,
                h_acc, u_acc):
    kc = pl.program_id(1)
    xb = xs_ref[...]
    hp = jnp.dot(xb, wg_ref[0], preferred_element_type=_F32)
    up = jnp.dot(xb, wu_ref[0], preferred_element_type=_F32)

    @pl.when(kc == 0)
    def _():
        h_acc[...] = hp
        u_acc[...] = up

    @pl.when(kc == 1)
    def _():
        h = h_acc[...] + hp
        u = u_acc[...] + up
        a = jax.nn.silu(h) * u * ws_ref[...]
        y_ref[...] = jnp.dot(a, wd_ref[0], preferred_element_type=_F32)


def _shared_kernel(x_ref, g_ref, u_ref, d_ref, out_ref):
    fs = pl.program_id(1)
    xb = x_ref[...].astype(jnp.bfloat16)
    h = jnp.dot(xb, g_ref[...], preferred_element_type=_F32)
    u = jnp.dot(xb, u_ref[...], preferred_element_type=_F32)
    a = (jax.nn.silu(h) * u).astype(jnp.bfloat16)
    val = jnp.dot(a, d_ref[...], preferred_element_type=_F32)

    @pl.when(fs == 0)
    def _():
        out_ref[...] = val

    @pl.when(fs > 0)
    def _():
        out_ref[...] += val


_CNW = 32
_CTW = _T // _CNW
_CCH = 16


def _sc_combine_body(y_hbm, sh_hbm, pos_hbm, out_hbm,
                     p1_v, p2_v, b1_v, b2_v, b3_v, sem1, sem2):
    wid = lax.axis_index("s") * 2 + lax.axis_index("c")
    t0 = wid * _CTW
    pltpu.sync_copy(pos_hbm.at[0, pl.ds(t0, _CTW)], p1_v)
    pltpu.sync_copy(pos_hbm.at[1, pl.ds(t0, _CTW)], p2_v)

    def chunk(c, carry):
        g1 = pltpu.async_copy(y_hbm.at[p1_v.at[pl.ds(c * _CCH, _CCH)]],
                              b1_v, sem1)
        g2 = pltpu.async_copy(y_hbm.at[p2_v.at[pl.ds(c * _CCH, _CCH)]],
                              b2_v, sem2)
        pltpu.sync_copy(sh_hbm.at[pl.ds(t0 + c * _CCH, _CCH), :], b3_v)
        g1.wait()
        g2.wait()
        for r in range(_CCH):
            def inner(j, carry2):
                col = j * 64
                for u in range(4):
                    sl = pl.ds(col + u * 16, 16)
                    b1_v[r, sl] = b1_v[r, sl] + b2_v[r, sl] + b3_v[r, sl]
                return carry2
            lax.fori_loop(0, _H // 64, inner, 0)
        pltpu.sync_copy(b1_v, out_hbm.at[pl.ds(t0 + c * _CCH, _CCH), :])
        return carry

    lax.fori_loop(0, _CTW // _CCH, chunk, 0)


def kernel(hidden_states, gate_w, w_gate, w_up, w_down, sw_gate, sw_up, sw_down):
    x = hidden_states
    T, H = x.shape
    E = gate_w.shape[1]
    F = w_gate.shape[2]
    DS = sw_gate.shape[1]

    logits = pl.pallas_call(
        _logits_kernel,
        grid=(T // 256,),
        in_specs=[
            pl.BlockSpec((256, H), lambda i: (i, 0)),
            pl.BlockSpec((H, E), lambda i: (0, 0)),
        ],
        out_specs=pl.BlockSpec((256, E), lambda i: (i, 0)),
        out_shape=jax.ShapeDtypeStruct((T, E), _F32),
    )(x, gate_w)

    bf16 = jnp.bfloat16
    fs = 1408 if DS % 1408 == 0 else DS
    shared = pl.pallas_call(
        _shared_kernel,
        grid=(T // 512, DS // fs),
        in_specs=[
            pl.BlockSpec((512, H), lambda i, j: (i, 0)),
            pl.BlockSpec((H, fs), lambda i, j: (0, j)),
            pl.BlockSpec((H, fs), lambda i, j: (0, j)),
            pl.BlockSpec((fs, H), lambda i, j: (j, 0)),
        ],
        out_specs=pl.BlockSpec((512, H), lambda i, j: (i, 0)),
        out_shape=jax.ShapeDtypeStruct((T, H), _F32),
    )(x, sw_gate.astype(bf16), sw_up.astype(bf16), sw_down.astype(bf16))

    route = pl.kernel(
        _sc_route_body,
        out_type=[
            jax.ShapeDtypeStruct((_NTOT, H), _F32),
            jax.ShapeDtypeStruct((_NTOT,), _F32),
            jax.ShapeDtypeStruct((_NBLK,), _I32),
            jax.ShapeDtypeStruct((2, T), _I32),
        ],
        mesh=plsc.VectorSubcoreMesh(core_axis_name="c", subcore_axis_name="s",
                                    num_cores=1),
        scratch_types=[
            pltpu.VMEM((_E, _TW), _F32),
            pltpu.VMEM((_TW,), _I32),
            pltpu.VMEM((_TW,), _I32),
            pltpu.VMEM((_TW,), _F32),
            pltpu.VMEM((_TW,), _F32),
            pltpu.VMEM((16,), _I32),
            pltpu.VMEM((16,), _I32),
            pltpu.VMEM((_NW * 16,), _I32),
            pltpu.VMEM((2 * _TW,), _I32),
            pltpu.VMEM((2, 128), _I32),
            pltpu.VMEM((2 * _TW,), _I32),
            pltpu.VMEM((2 * _TW,), _F32),
            pltpu.VMEM((_TW,), _I32),
            pltpu.VMEM((_TW,), _I32),
            pltpu.VMEM((_NBLK,), _I32),
            pltpu.VMEM((_RW,), _I32),
            pltpu.VMEM((_RW,), _F32),
            pltpu.VMEM((32,), _I32),
            pltpu.VMEM((_RW,), _I32),
            pltpu.VMEM((_GCH, H), _F32),
            pltpu.VMEM((_GCH, H), _F32),
            pltpu.VMEM_SHARED((_NW * 16,), _I32),
            pltpu.VMEM_SHARED((_NTOT,), _I32),
            pltpu.VMEM_SHARED((_NTOT,), _F32),
            pltpu.SemaphoreType.DMA,
            pltpu.SemaphoreType.DMA,
        ],
    )
    xs, ws, be, pos = route(logits.T, x)

    kcn = 2
    kch = H // kcn
    y = pl.pallas_call(
        _gmm_kernel,
        grid_spec=pltpu.PrefetchScalarGridSpec(
            num_scalar_prefetch=1,
            grid=(_NBLK, kcn),
            in_specs=[
                pl.BlockSpec((_BT, kch), lambda b, k, be: (b, k)),
                pl.BlockSpec((_BT, 1), lambda b, k, be: (b, 0)),
                pl.BlockSpec((1, kch, F), lambda b, k, be: (be[b], k, 0)),
                pl.BlockSpec((1, kch, F), lambda b, k, be: (be[b], k, 0)),
                pl.BlockSpec((1, F, H), lambda b, k, be: (be[b], 0, 0)),
            ],
            out_specs=pl.BlockSpec((_BT, H), lambda b, k, be: (b, 0)),
            scratch_shapes=[
                pltpu.VMEM((_BT, F), _F32),
                pltpu.VMEM((_BT, F), _F32),
            ],
        ),
        out_shape=jax.ShapeDtypeStruct((_NTOT, H), _F32),
    )(be, xs, ws.reshape(_NTOT, 1), w_gate, w_up, w_down)

    comb = pl.kernel(
        _sc_combine_body,
        out_type=jax.ShapeDtypeStruct((T, H), _F32),
        mesh=plsc.VectorSubcoreMesh(core_axis_name="c", subcore_axis_name="s"),
        scratch_types=[
            pltpu.VMEM((_CTW,), _I32),
            pltpu.VMEM((_CTW,), _I32),
            pltpu.VMEM((_CCH, H), _F32),
            pltpu.VMEM((_CCH, H), _F32),
            pltpu.VMEM((_CCH, H), _F32),
            pltpu.SemaphoreType.DMA,
            pltpu.SemaphoreType.DMA,
        ],
    )
    out = comb(y, shared, pos)
    return out


# speedup vs baseline: 1.4686x; 1.0499x over previous
import functools

import jax
import jax.numpy as jnp
from jax import lax
from jax.experimental import pallas as pl
from jax.experimental.pallas import tpu as pltpu
from jax.experimental.pallas import tpu_sc as plsc

_F32 = jnp.float32
_I32 = jnp.int32

_E = 16
_NG = 4
_GS = _E // _NG
_T = 2048
_H = 2048
_BT = 256
_NBLK = 32
_NTOT = _NBLK * _BT
_NW = 16
_TW = _T // _NW
_RW = _NTOT // _NW
_GCH = 16


def _iota16():
    return lax.broadcasted_iota(_I32, (16,), 0)


def _splat(val, dtype=_I32):
    return jnp.full((16,), val, dtype)


def _logits_kernel(x_ref, gw_ref, out_ref, x32_ref):
    xb = x_ref[...]
    out_ref[...] = jnp.dot(xb, gw_ref[...], preferred_element_type=_F32)
    h2 = xb.shape[1] // 2
    ai = pltpu.bitcast(xb[:, :h2], _I32)
    bi = pltpu.bitcast(xb[:, h2:], _I32)
    ha = lax.shift_right_logical(
        ai + ((lax.shift_right_logical(ai, 16) & 1) + 0x7FFF), 16)
    hb = lax.shift_right_logical(
        bi + ((lax.shift_right_logical(bi, 16) & 1) + 0x7FFF), 16)
    x32_ref[...] = ha | lax.shift_left(hb, 16)


def _sc_route_body(lg_hbm, x_hbm, xs_hbm, ws_hbm, be_hbm, pos_hbm,
                   lg_v, e1i_v, e2i_v, w1_v, w2_v, cur_v, bc_v, hall_v,
                   spos_v, spos2_v, stok_v, swv_v, pos1_v, pos2_v, bev_v,
                   zz_v, wsz_v, ptmp_v, sidx_v, rows0_v, rows1_v,
                   hist_sh, stok_sh, wsrt_sh, sem0, sem1):
    wid = lax.axis_index("s")
    t0 = wid * _TW
    lanes = _iota16()

    ptmp_v[pl.ds(0, 16)] = jnp.zeros((16,), _I32)

    def psum16(x):
        acc = x
        for k in (1, 2, 4, 8):
            ptmp_v[pl.ds(16, 16)] = acc
            acc = acc + ptmp_v[pl.ds(16 - k, 16)]
        return acc

    pltpu.sync_copy(lg_hbm.at[:, pl.ds(t0, _TW)], lg_v)

    def route_group(g, accs):
        s = [lg_v[e, pl.ds(g * 16, 16)] for e in range(_E)]
        m = s[0]
        for e in range(1, _E):
            m = jnp.maximum(m, s[e])
        ex = [jnp.exp(s[e] - m) for e in range(_E)]
        tot = ex[0]
        for e in range(1, _E):
            tot = tot + ex[e]
        inv = 1.0 / tot
        sc = [ex[e] * inv for e in range(_E)]
        gm = []
        for gi in range(_NG):
            v = sc[gi * _GS]
            for j in range(1, _GS):
                v = jnp.maximum(v, sc[gi * _GS + j])
            gm.append(v)
        v1 = gm[0]
        i1 = jnp.zeros((16,), _I32)
        for gi in range(1, _NG):
            better = gm[gi] > v1
            v1 = jnp.where(better, gm[gi], v1)
            i1 = jnp.where(better, gi, i1)
        v2 = _splat(-1e30, _F32)
        i2 = _splat(-1, _I32)
        for gi in range(_NG):
            cand = (gm[gi] > v2) & (i1 != gi)
            v2 = jnp.where(cand, gm[gi], v2)
            i2 = jnp.where(cand, gi, i2)
        ms = []
        e1v = _splat(-1e30, _F32)
        e1i = jnp.zeros((16,), _I32)
        for e in range(_E):
            ge = e // _GS
            allowed = (i1 == ge) | (i2 == ge)
            mse = jnp.where(allowed, sc[e], 0.0)
            ms.append(mse)
            better = mse > e1v
            e1v = jnp.where(better, mse, e1v)
            e1i = jnp.where(better, e, e1i)
        e2v = _splat(-1e30, _F32)
        e2i = jnp.zeros((16,), _I32)
        for e in range(_E):
            cand = (ms[e] > e2v) & (e1i != e)
            e2v = jnp.where(cand, ms[e], e2v)
            e2i = jnp.where(cand, e, e2i)
        den = e1v + e2v + 1e-20
        w1 = e1v / den
        w2 = e2v / den
        e1i_v[pl.ds(g * 16, 16)] = e1i
        e2i_v[pl.ds(g * 16, 16)] = e2i
        w1_v[pl.ds(g * 16, 16)] = w1
        w2_v[pl.ds(g * 16, 16)] = w2
        return tuple(
            accs[e]
            + jnp.where(e1i == e, 1, 0)
            + jnp.where(e2i == e, 1, 0)
            for e in range(_E)
        )

    accs = lax.fori_loop(0, _TW // 16, route_group,
                         tuple(jnp.zeros((16,), _I32) for _ in range(_E)))
    hist = jnp.zeros((16,), _I32)
    for e in range(_E):
        tot_e = psum16(accs[e])[15]
        hist = hist + jnp.where(lanes == e, tot_e, 0)

    bc_v[...] = hist
    pltpu.sync_copy(bc_v, hist_sh.at[pl.ds(wid * 16, 16)])
    zi = jnp.zeros((16,), _I32)
    zf = jnp.zeros((16,), _F32)
    for j in range(_RW // 16):
        zz_v[pl.ds(j * 16, 16)] = zi
        wsz_v[pl.ds(j * 16, 16)] = zf
    pltpu.sync_copy(zz_v, stok_sh.at[pl.ds(wid * _RW, _RW)])
    pltpu.sync_copy(wsz_v, wsrt_sh.at[pl.ds(wid * _RW, _RW)])
    plsc.subcore_barrier()
    pltpu.sync_copy(hist_sh, hall_v)
    totals = jnp.zeros((16,), _I32)
    prefix = jnp.zeros((16,), _I32)
    for w in range(_NW):
        row = hall_v[pl.ds(w * 16, 16)]
        totals = totals + row
        prefix = prefix + row * (jnp.int32(w) < wid).astype(_I32)
    pad = (totals + (_BT - 1)) & _splat(-_BT)
    bb_incl = psum16(pad)
    block_base = bb_incl - pad
    cur_v[...] = block_base + prefix

    @pl.when(wid == 0)
    def _():
        endvec = lax.shift_right_logical(bb_incl, 8)
        blk0 = lanes
        blk1 = lanes + 16
        acc0 = jnp.zeros((16,), _I32)
        acc1 = jnp.zeros((16,), _I32)
        for e in range(_E):
            endv = endvec[e]
            acc0 = acc0 + jnp.where(blk0 >= endv, 1, 0)
            acc1 = acc1 + jnp.where(blk1 >= endv, 1, 0)
        bev_v[pl.ds(0, 16)] = jnp.minimum(acc0, _E - 1)
        bev_v[pl.ds(16, 16)] = jnp.minimum(acc1, _E - 1)
        pltpu.sync_copy(bev_v, be_hbm)

    def place(ids, pos_init):
        pos = pos_init
        for e in range(_E):
            msk = ids == e
            rank = psum16(jnp.where(msk, 1, 0))
            cnt = rank[15]
            cur = cur_v[...]
            base = cur[e]
            pos = jnp.where(msk, base + rank - 1, pos)
            cur_v[...] = cur + jnp.where(lanes == e, cnt, 0)
        return pos

    def place_group(g, carry):
        ids1 = e1i_v[pl.ds(g * 16, 16)]
        ids2 = e2i_v[pl.ds(g * 16, 16)]
        tok = t0 + g * 16 + lanes
        p1 = place(ids1, jnp.zeros((16,), _I32))
        p2 = place(ids2, jnp.zeros((16,), _I32))
        pos1_v[pl.ds(g * 16, 16)] = p1
        pos2_v[pl.ds(g * 16, 16)] = p2
        spos_v[pl.ds(g * 32, 16)] = p1
        spos_v[pl.ds(g * 32 + 16, 16)] = p2
        stok_v[pl.ds(g * 32, 16)] = tok
        stok_v[pl.ds(g * 32 + 16, 16)] = tok
        swv_v[pl.ds(g * 32, 16)] = w1_v[pl.ds(g * 16, 16)]
        swv_v[pl.ds(g * 32 + 16, 16)] = w2_v[pl.ds(g * 16, 16)]
        return carry

    lax.fori_loop(0, _TW // 16, place_group, 0)

    for j in range(16):
        spos2_v[j // 8, pl.ds((j % 8) * 16, 16)] = spos_v[pl.ds(j * 16, 16)]
    for k in range(2):
        pltpu.sync_copy(stok_v.at[pl.ds(k * 128, 128)],
                        stok_sh.at[spos2_v.at[k]])
        pltpu.sync_copy(swv_v.at[pl.ds(k * 128, 128)],
                        wsrt_sh.at[spos2_v.at[k]])
    pltpu.sync_copy(pos1_v, pos_hbm.at[0, pl.ds(t0, _TW)])
    pltpu.sync_copy(pos2_v, pos_hbm.at[1, pl.ds(t0, _TW)])
    plsc.subcore_barrier()

    r0 = wid * _RW
    pltpu.sync_copy(stok_sh.at[pl.ds(r0, _RW)], sidx_v)
    pltpu.sync_copy(wsrt_sh.at[pl.ds(r0, _RW)], ws_hbm.at[pl.ds(r0, _RW)])
    bufs = (rows0_v, rows1_v)
    sems = (sem0, sem1)
    nch = _RW // _GCH
    pend = []
    for c in range(2):
        pend.append(pltpu.async_copy(
            x_hbm.at[sidx_v.at[pl.ds(c * _GCH, _GCH)]],
            bufs[c % 2], sems[c % 2]))
    for c in range(nch):
        pend[c].wait()
        pltpu.sync_copy(bufs[c % 2],
                        xs_hbm.at[pl.ds(r0 + c * _GCH, _GCH), :])
        if c + 2 < nch:
            pend.append(pltpu.async_copy(
                x_hbm.at[sidx_v.at[pl.ds((c + 2) * _GCH, _GCH)]],
                bufs[c % 2], sems[c % 2]))


def _gmm_kernel(be_ref, xs_ref, ws_ref, wg_ref, wu_ref, wd_ref, y_ref,
                h_acc, u_acc):
    kc = pl.program_id(1)
    xsv = xs_ref[...]
    xb = pltpu.bitcast(
        jnp.where(kc == 0, lax.shift_left(xsv, 16),
                  xsv & jnp.int32(-65536)), _F32)
    hp = jnp.dot(xb, wg_ref[0], preferred_element_type=_F32)
    up = jnp.dot(xb, wu_ref[0], preferred_element_type=_F32)

    @pl.when(kc == 0)
    def _():
        h_acc[...] = hp
        u_acc[...] = up

    @pl.when(kc == 1)
    def _():
        h = h_acc[...] + hp
        u = u_acc[...] + up
        a = jax.nn.silu(h) * u * ws_ref[...]
        y_ref[...] = jnp.dot(a, wd_ref[0], preferred_element_type=_F32)


def _shared_kernel(x_ref, g_ref, u_ref, d_ref, out_ref):
    fs = pl.program_id(1)
    xb = x_ref[...].astype(jnp.bfloat16)
    h = jnp.dot(xb, g_ref[...], preferred_element_type=_F32)
    u = jnp.dot(xb, u_ref[...], preferred_element_type=_F32)
    a = (jax.nn.silu(h) * u).astype(jnp.bfloat16)
    val = jnp.dot(a, d_ref[...], preferred_element_type=_F32)

    @pl.when(fs == 0)
    def _():
        out_ref[...] = val

    @pl.when(fs > 0)
    def _():
        out_ref[...] += val


_CNW = 32
_CTW = _T // _CNW
_CCH = 16


def _sc_combine_body(y_hbm, sh_hbm, pos_hbm, out_hbm,
                     p1_v, p2_v, b1_v, b2_v, b3_v, sem1, sem2):
    wid = lax.axis_index("s") * 2 + lax.axis_index("c")
    t0 = wid * _CTW
    pltpu.sync_copy(pos_hbm.at[0, pl.ds(t0, _CTW)], p1_v)
    pltpu.sync_copy(pos_hbm.at[1, pl.ds(t0, _CTW)], p2_v)

    def chunk(c, carry):
        g1 = pltpu.async_copy(y_hbm.at[p1_v.at[pl.ds(c * _CCH, _CCH)]],
                              b1_v, sem1)
        g2 = pltpu.async_copy(y_hbm.at[p2_v.at[pl.ds(c * _CCH, _CCH)]],
                              b2_v, sem2)
        pltpu.sync_copy(sh_hbm.at[pl.ds(t0 + c * _CCH, _CCH), :], b3_v)
        g1.wait()
        g2.wait()
        for r in range(_CCH):
            def inner(j, carry2):
                col = j * 64
                for u in range(4):
                    sl = pl.ds(col + u * 16, 16)
                    b1_v[r, sl] = b1_v[r, sl] + b2_v[r, sl] + b3_v[r, sl]
                return carry2
            lax.fori_loop(0, _H // 64, inner, 0)
        pltpu.sync_copy(b1_v, out_hbm.at[pl.ds(t0 + c * _CCH, _CCH), :])
        return carry

    lax.fori_loop(0, _CTW // _CCH, chunk, 0)


def kernel(hidden_states, gate_w, w_gate, w_up, w_down, sw_gate, sw_up, sw_down):
    x = hidden_states
    T, H = x.shape
    E = gate_w.shape[1]
    F = w_gate.shape[2]
    DS = sw_gate.shape[1]

    logits, x32 = pl.pallas_call(
        _logits_kernel,
        grid=(T // 256,),
        in_specs=[
            pl.BlockSpec((256, H), lambda i: (i, 0)),
            pl.BlockSpec((H, E), lambda i: (0, 0)),
        ],
        out_specs=[
            pl.BlockSpec((256, E), lambda i: (i, 0)),
            pl.BlockSpec((256, H // 2), lambda i: (i, 0)),
        ],
        out_shape=[
            jax.ShapeDtypeStruct((T, E), _F32),
            jax.ShapeDtypeStruct((T, H // 2), _I32),
        ],
    )(x, gate_w)

    bf16 = jnp.bfloat16
    fs = 1408 if DS % 1408 == 0 else DS
    shared = pl.pallas_call(
        _shared_kernel,
        grid=(T // 512, DS // fs),
        in_specs=[
            pl.BlockSpec((512, H), lambda i, j: (i, 0)),
            pl.BlockSpec((H, fs), lambda i, j: (0, j)),
            pl.BlockSpec((H, fs), lambda i, j: (0, j)),
            pl.BlockSpec((fs, H), lambda i, j: (j, 0)),
        ],
        out_specs=pl.BlockSpec((512, H), lambda i, j: (i, 0)),
        out_shape=jax.ShapeDtypeStruct((T, H), _F32),
    )(x, sw_gate.astype(bf16), sw_up.astype(bf16), sw_down.astype(bf16))

    route = pl.kernel(
        _sc_route_body,
        out_type=[
            jax.ShapeDtypeStruct((_NTOT, H // 2), _I32),
            jax.ShapeDtypeStruct((_NTOT,), _F32),
            jax.ShapeDtypeStruct((_NBLK,), _I32),
            jax.ShapeDtypeStruct((2, T), _I32),
        ],
        mesh=plsc.VectorSubcoreMesh(core_axis_name="c", subcore_axis_name="s",
                                    num_cores=1),
        scratch_types=[
            pltpu.VMEM((_E, _TW), _F32),
            pltpu.VMEM((_TW,), _I32),
            pltpu.VMEM((_TW,), _I32),
            pltpu.VMEM((_TW,), _F32),
            pltpu.VMEM((_TW,), _F32),
            pltpu.VMEM((16,), _I32),
            pltpu.VMEM((16,), _I32),
            pltpu.VMEM((_NW * 16,), _I32),
            pltpu.VMEM((2 * _TW,), _I32),
            pltpu.VMEM((2, 128), _I32),
            pltpu.VMEM((2 * _TW,), _I32),
            pltpu.VMEM((2 * _TW,), _F32),
            pltpu.VMEM((_TW,), _I32),
            pltpu.VMEM((_TW,), _I32),
            pltpu.VMEM((_NBLK,), _I32),
            pltpu.VMEM((_RW,), _I32),
            pltpu.VMEM((_RW,), _F32),
            pltpu.VMEM((32,), _I32),
            pltpu.VMEM((_RW,), _I32),
            pltpu.VMEM((_GCH, _H // 2), _I32),
            pltpu.VMEM((_GCH, _H // 2), _I32),
            pltpu.VMEM_SHARED((_NW * 16,), _I32),
            pltpu.VMEM_SHARED((_NTOT,), _I32),
            pltpu.VMEM_SHARED((_NTOT,), _F32),
            pltpu.SemaphoreType.DMA,
            pltpu.SemaphoreType.DMA,
        ],
    )
    xs32, ws, be, pos = route(logits.T, x32)

    kcn = 2
    kch = H // kcn
    y = pl.pallas_call(
        _gmm_kernel,
        grid_spec=pltpu.PrefetchScalarGridSpec(
            num_scalar_prefetch=1,
            grid=(_NBLK, kcn),
            in_specs=[
                pl.BlockSpec((_BT, H // 2), lambda b, k, be: (b, 0)),
                pl.BlockSpec((_BT, 1), lambda b, k, be: (b, 0)),
                pl.BlockSpec((1, kch, F), lambda b, k, be: (be[b], k, 0)),
                pl.BlockSpec((1, kch, F), lambda b, k, be: (be[b], k, 0)),
                pl.BlockSpec((1, F, H), lambda b, k, be: (be[b], 0, 0)),
            ],
            out_specs=pl.BlockSpec((_BT, H), lambda b, k, be: (b, 0)),
            scratch_shapes=[
                pltpu.VMEM((_BT, F), _F32),
                pltpu.VMEM((_BT, F), _F32),
            ],
        ),
        out_shape=jax.ShapeDtypeStruct((_NTOT, H), _F32),
    )(be, xs32, ws.reshape(_NTOT, 1), w_gate, w_up, w_down)

    comb = pl.kernel(
        _sc_combine_body,
        out_type=jax.ShapeDtypeStruct((T, H), _F32),
        mesh=plsc.VectorSubcoreMesh(core_axis_name="c", subcore_axis_name="s"),
        scratch_types=[
            pltpu.VMEM((_CTW,), _I32),
            pltpu.VMEM((_CTW,), _I32),
            pltpu.VMEM((_CCH, H), _F32),
            pltpu.VMEM((_CCH, H), _F32),
            pltpu.VMEM((_CCH, H), _F32),
            pltpu.SemaphoreType.DMA,
            pltpu.SemaphoreType.DMA,
        ],
    )
    out = comb(y, shared, pos)
    return out
